# baseline (speedup 1.0000x reference)
import functools

import jax
import jax.numpy as jnp
from jax import lax
from jax.experimental import pallas as pl
from jax.experimental.pallas import tpu as pltpu
from jax.experimental.pallas import tpu_sc as plsc

NC = 2
NS = 16
LANES = 16

CH = 128
ZROWS = 16


STAGE = 8
ST0 = 13
ST1 = 7


def _make_spmm(n, epad, h):
    nz = n // ZROWS
    napad = n + 8
    assert NS * STAGE * (ST0 + ST1) * CH == epad

    mesh = plsc.VectorSubcoreMesh(core_axis_name="c", subcore_axis_name="s")

    @functools.partial(
        pl.kernel,
        mesh=mesh,
        out_type=jax.ShapeDtypeStruct((NC * n, h), jnp.float32),
        scratch_types=[
            pltpu.VMEM_SHARED((napad, h), jnp.float32),
            pltpu.VMEM((STAGE, CH), jnp.int32),
            pltpu.VMEM((STAGE, CH), jnp.int32),
            pltpu.VMEM((CH, h), jnp.float32),
            pltpu.VMEM((CH, h), jnp.float32),
            pltpu.VMEM((ZROWS, h), jnp.float32),
            pltpu.SemaphoreType.DMA,
            pltpu.SemaphoreType.DMA,
            pltpu.SemaphoreType.DMA,
        ],
    )
    def spmm(support_hbm, src_hbm, dst_hbm, out_hbm, acc_sh, src_v, dst_v,
             rows_a, rows_b, zero_v, gsem, ssem_a, ssem_b):
        c = lax.axis_index("c")
        s = lax.axis_index("s")

        zvec = jnp.zeros((LANES,), jnp.float32)

        @pl.loop(0, ZROWS)
        def _(r):
            @pl.loop(0, h, step=LANES)
            def _(c0):
                zero_v[r, pl.ds(c0, LANES)] = zvec

        @pl.loop(s, nz, step=NS)
        def _(kk):
            off = pl.multiple_of(kk * ZROWS, 8)
            pltpu.sync_copy(zero_v, acc_sh.at[pl.ds(off, ZROWS)])

        plsc.subcore_barrier()

        bufs = ((rows_a, ssem_a), (rows_b, ssem_b))

        nst = jnp.where(c == 0, ST0, ST1)
        base = jnp.where(c == 0, s * ST0 * STAGE,
                         (NS * ST0 + s * ST1) * STAGE)

        @pl.loop(0, nst)
        def _(st):
            ibase = pl.multiple_of(base + st * STAGE, 8)
            pltpu.sync_copy(src_hbm.at[pl.ds(ibase, STAGE)], src_v)
            pltpu.sync_copy(dst_hbm.at[pl.ds(ibase, STAGE)], dst_v)

            @pl.loop(0, STAGE, step=2)
            def _(jj):
                for b, (rbuf, ssem) in enumerate(bufs):
                    j = jj + b

                    @pl.when(jj > 0)
                    def _():
                        pltpu.make_async_copy(
                            rbuf, acc_sh.at[dst_v.at[j]], ssem).wait()

                    pltpu.async_copy(support_hbm.at[src_v.at[j]], rbuf,
                                     gsem).wait()
                    pltpu.async_copy(rbuf, acc_sh.at[dst_v.at[j]], ssem,
                                     add=True)

            for b, (rbuf, ssem) in enumerate(bufs):
                pltpu.make_async_copy(
                    rbuf, acc_sh.at[dst_v.at[STAGE - 2 + b]], ssem).wait()

        plsc.subcore_barrier()

        @pl.loop(s, nz, step=NS)
        def _(kk):
            off = pl.multiple_of(kk * ZROWS, 8)
            off2 = pl.multiple_of(c * n + kk * ZROWS, 8)
            pltpu.sync_copy(acc_sh.at[pl.ds(off, ZROWS)],
                            out_hbm.at[pl.ds(off2, ZROWS)])

    return spmm


def _matmul_body(x_ref, w_ref, o_ref):
    o_ref[...] = lax.dot_general(
        x_ref[...], w_ref[...], (((1,), (0,)), ((), ())),
        preferred_element_type=jnp.float32, precision=lax.Precision.HIGHEST)


def _mid_body(p_ref, b_ref, w_ref, o_ref):
    hval = jnp.maximum(p_ref[0] + p_ref[1] + b_ref[...], 0.0)
    o_ref[...] = lax.dot_general(
        hval, w_ref[...], (((1,), (0,)), ((), ())),
        preferred_element_type=jnp.float32, precision=lax.Precision.HIGHEST)


def _final_body(p_ref, b_ref, o_ref):
    o_ref[...] = jnp.maximum(p_ref[0] + p_ref[1] + b_ref[...], 0.0)


def kernel(x, adj, W1, b1, W2, b2):
    n, h = x.shape
    e = adj.shape[1]
    f = W2.shape[1]

    quantum = NC * NS * CH * 8
    epad = ((e + quantum - 1) // quantum) * quantum
    src = jnp.concatenate(
        [adj[0], jnp.zeros((epad - e,), jnp.int32)]).reshape(epad // CH, CH)
    dst = jnp.concatenate(
        [adj[1], jnp.full((epad - e,), n, jnp.int32)]).reshape(epad // CH, CH)

    rb = 2000
    grid = (n // rb,)

    support1 = pl.pallas_call(
        _matmul_body,
        grid=grid,
        in_specs=[
            pl.BlockSpec((rb, h), lambda i: (i, 0)),
            pl.BlockSpec((h, h), lambda i: (0, 0)),
        ],
        out_specs=pl.BlockSpec((rb, h), lambda i: (i, 0)),
        out_shape=jax.ShapeDtypeStruct((n, h), jnp.float32),
    )(x, W1)

    spmm = _make_spmm(n, epad, h)
    p1 = spmm(support1, src, dst).reshape(NC, n, h)

    support2 = pl.pallas_call(
        _mid_body,
        grid=grid,
        in_specs=[
            pl.BlockSpec((NC, rb, h), lambda i: (0, i, 0)),
            pl.BlockSpec((1, h), lambda i: (0, 0)),
            pl.BlockSpec((h, f), lambda i: (0, 0)),
        ],
        out_specs=pl.BlockSpec((rb, f), lambda i: (i, 0)),
        out_shape=jax.ShapeDtypeStruct((n, f), jnp.float32),
    )(p1, b1.reshape(1, h), W2)

    p2 = spmm(support2, src, dst).reshape(NC, n, f)

    out = pl.pallas_call(
        _final_body,
        grid=grid,
        in_specs=[
            pl.BlockSpec((NC, rb, f), lambda i: (0, i, 0)),
            pl.BlockSpec((1, f), lambda i: (0, 0)),
        ],
        out_specs=pl.BlockSpec((rb, f), lambda i: (i, 0)),
        out_shape=jax.ShapeDtypeStruct((n, f), jnp.float32),
    )(p2, b2.reshape(1, f))

    return out


# speedup vs baseline: 1.0613x; 1.0613x over previous
import functools

import jax
import jax.numpy as jnp
from jax import lax
from jax.experimental import pallas as pl
from jax.experimental.pallas import tpu as pltpu
from jax.experimental.pallas import tpu_sc as plsc

NC = 2
NS = 16
LANES = 16

CH = 128
ZROWS = 16


STAGE = 32
ST0 = 4
ST1 = 1


def _make_spmm(n, epad, h):
    nz = n // ZROWS
    napad = n + 8
    assert NS * STAGE * (ST0 + ST1) * CH == epad

    mesh = plsc.VectorSubcoreMesh(core_axis_name="c", subcore_axis_name="s")

    @functools.partial(
        pl.kernel,
        mesh=mesh,
        out_type=jax.ShapeDtypeStruct((NC * n, h), jnp.float32),
        scratch_types=[
            pltpu.VMEM_SHARED((napad, h), jnp.float32),
            pltpu.VMEM((STAGE, CH), jnp.int32),
            pltpu.VMEM((STAGE, CH), jnp.int32),
            pltpu.VMEM((CH, h), jnp.float32),
            pltpu.VMEM((CH, h), jnp.float32),
            pltpu.VMEM((ZROWS, h), jnp.float32),
            pltpu.SemaphoreType.DMA,
            pltpu.SemaphoreType.DMA,
            pltpu.SemaphoreType.DMA,
        ],
    )
    def spmm(support_hbm, src_hbm, dst_hbm, out_hbm, acc_sh, src_v, dst_v,
             rows_a, rows_b, zero_v, gsem, ssem_a, ssem_b):
        c = lax.axis_index("c")
        s = lax.axis_index("s")

        zvec = jnp.zeros((LANES,), jnp.float32)

        @pl.loop(0, ZROWS)
        def _(r):
            @pl.loop(0, h, step=LANES)
            def _(c0):
                zero_v[r, pl.ds(c0, LANES)] = zvec

        @pl.loop(s, nz, step=NS)
        def _(kk):
            off = pl.multiple_of(kk * ZROWS, 8)
            pltpu.sync_copy(zero_v, acc_sh.at[pl.ds(off, ZROWS)])

        plsc.subcore_barrier()

        bufs = ((rows_a, ssem_a), (rows_b, ssem_b))

        nst = jnp.where(c == 0, ST0, ST1)
        base = jnp.where(c == 0, s * ST0 * STAGE,
                         (NS * ST0 + s * ST1) * STAGE)

        @pl.loop(0, nst)
        def _(st):
            ibase = pl.multiple_of(base + st * STAGE, 8)
            pltpu.sync_copy(src_hbm.at[pl.ds(ibase, STAGE)], src_v)
            pltpu.sync_copy(dst_hbm.at[pl.ds(ibase, STAGE)], dst_v)

            @pl.loop(0, STAGE, step=2)
            def _(jj):
                for b, (rbuf, ssem) in enumerate(bufs):
                    j = jj + b

                    @pl.when(jj > 0)
                    def _():
                        pltpu.make_async_copy(
                            rbuf, acc_sh.at[dst_v.at[j]], ssem).wait()

                    pltpu.async_copy(support_hbm.at[src_v.at[j]], rbuf,
                                     gsem).wait()
                    pltpu.async_copy(rbuf, acc_sh.at[dst_v.at[j]], ssem,
                                     add=True)

            for b, (rbuf, ssem) in enumerate(bufs):
                pltpu.make_async_copy(
                    rbuf, acc_sh.at[dst_v.at[STAGE - 2 + b]], ssem).wait()

        plsc.subcore_barrier()

        @pl.loop(s, nz, step=NS)
        def _(kk):
            off = pl.multiple_of(kk * ZROWS, 8)
            off2 = pl.multiple_of(c * n + kk * ZROWS, 8)
            pltpu.sync_copy(acc_sh.at[pl.ds(off, ZROWS)],
                            out_hbm.at[pl.ds(off2, ZROWS)])

    return spmm


def _matmul_body(x_ref, w_ref, o_ref):
    o_ref[...] = lax.dot_general(
        x_ref[...], w_ref[...], (((1,), (0,)), ((), ())),
        preferred_element_type=jnp.float32, precision=lax.Precision.HIGHEST)


def _mid_body(p_ref, b_ref, w_ref, o_ref):
    hval = jnp.maximum(p_ref[0] + p_ref[1] + b_ref[...], 0.0)
    o_ref[...] = lax.dot_general(
        hval, w_ref[...], (((1,), (0,)), ((), ())),
        preferred_element_type=jnp.float32, precision=lax.Precision.HIGHEST)


def _final_body(p_ref, b_ref, o_ref):
    o_ref[...] = jnp.maximum(p_ref[0] + p_ref[1] + b_ref[...], 0.0)


def kernel(x, adj, W1, b1, W2, b2):
    n, h = x.shape
    e = adj.shape[1]
    f = W2.shape[1]

    quantum = NC * NS * CH * 8
    epad = ((e + quantum - 1) // quantum) * quantum
    src = jnp.concatenate(
        [adj[0], jnp.zeros((epad - e,), jnp.int32)]).reshape(epad // CH, CH)
    dst = jnp.concatenate(
        [adj[1], jnp.full((epad - e,), n, jnp.int32)]).reshape(epad // CH, CH)

    rb = 2000
    grid = (n // rb,)

    support1 = pl.pallas_call(
        _matmul_body,
        grid=grid,
        in_specs=[
            pl.BlockSpec((rb, h), lambda i: (i, 0)),
            pl.BlockSpec((h, h), lambda i: (0, 0)),
        ],
        out_specs=pl.BlockSpec((rb, h), lambda i: (i, 0)),
        out_shape=jax.ShapeDtypeStruct((n, h), jnp.float32),
    )(x, W1)

    spmm = _make_spmm(n, epad, h)
    p1 = spmm(support1, src, dst).reshape(NC, n, h)

    support2 = pl.pallas_call(
        _mid_body,
        grid=grid,
        in_specs=[
            pl.BlockSpec((NC, rb, h), lambda i: (0, i, 0)),
            pl.BlockSpec((1, h), lambda i: (0, 0)),
            pl.BlockSpec((h, f), lambda i: (0, 0)),
        ],
        out_specs=pl.BlockSpec((rb, f), lambda i: (i, 0)),
        out_shape=jax.ShapeDtypeStruct((n, f), jnp.float32),
    )(p1, b1.reshape(1, h), W2)

    p2 = spmm(support2, src, dst).reshape(NC, n, f)

    out = pl.pallas_call(
        _final_body,
        grid=grid,
        in_specs=[
            pl.BlockSpec((NC, rb, f), lambda i: (0, i, 0)),
            pl.BlockSpec((1, f), lambda i: (0, 0)),
        ],
        out_specs=pl.BlockSpec((rb, f), lambda i: (i, 0)),
        out_shape=jax.ShapeDtypeStruct((n, f), jnp.float32),
    )(p2, b2.reshape(1, f))

    return out
